# Initial kernel scaffold; baseline (speedup 1.0000x reference)
import jax
import jax.numpy as jnp
from jax.experimental import pallas as pl


def kernel(h, vn_h, segment_ids, W, b):
    raise NotImplementedError("write your pallas kernel here")


# speedup vs baseline: 3.5903x; 3.5903x over previous
import functools

import jax
import jax.numpy as jnp
from jax import lax
from jax.experimental import pallas as pl
from jax.experimental.pallas import tpu as pltpu
from jax.experimental.pallas import tpu_sc as plsc

N = 100000
D = 128
B = 64

NC = 2
NS = 16
NW = NC * NS
CH = 400
NCHUNK = N // CH
CH_PER_W = -(-NCHUNK // NW)
NLANE = 16
NJ = D // NLANE
BND_PAD = 128

_mesh = plsc.VectorSubcoreMesh(
    core_axis_name="c", subcore_axis_name="s", num_cores=NC, num_subcores=NS
)


def _worker_id():
    return lax.axis_index("s") * NC + lax.axis_index("c")


def _load_bounds(bnd_v):
    groups = [bnd_v[pl.ds(NLANE * g, NLANE)] for g in range((B + NLANE) // NLANE)]
    return [groups[s // NLANE][s % NLANE] for s in range(B + 1)]


def _seg_partial_body(h_hbm, bnd_hbm, part_hbm, bnd_v, buf, acc):
    wid = _worker_id()
    pltpu.sync_copy(bnd_hbm, bnd_v)
    bnd = _load_bounds(bnd_v)

    zero = jnp.zeros((NLANE,), jnp.float32)
    for r in range(B):
        for j in range(NJ):
            acc[r, pl.ds(NLANE * j, NLANE)] = zero

    def chunk_body(m, carry):
        t = wid + m * NW

        @pl.when(t < NCHUNK)
        def _():
            c0 = t * CH
            pltpu.sync_copy(h_hbm.at[pl.ds(c0, CH)], buf)
            for s in range(B):
                lo = jnp.maximum(bnd[s], c0)
                hi = jnp.minimum(bnd[s + 1], c0 + CH)

                @pl.when(lo < hi)
                def _():
                    def row_body(i, a8):
                        off = i - c0
                        return tuple(
                            a8[j] + buf[off, pl.ds(NLANE * j, NLANE)]
                            for j in range(NJ)
                        )

                    a8 = lax.fori_loop(lo, hi, row_body, (zero,) * NJ)
                    for j in range(NJ):
                        sl = pl.ds(NLANE * j, NLANE)
                        acc[s, sl] = acc[s, sl] + a8[j]

        return carry

    lax.fori_loop(0, CH_PER_W, chunk_body, 0)
    pltpu.sync_copy(acc, part_hbm.at[wid])


_seg_partial = pl.kernel(
    _seg_partial_body,
    out_type=jax.ShapeDtypeStruct((NW, B, D), jnp.float32),
    mesh=_mesh,
    scratch_types=[
        pltpu.VMEM((BND_PAD,), jnp.int32),
        pltpu.VMEM((CH, D), jnp.float32),
        pltpu.VMEM((B, D), jnp.float32),
    ],
)


def _fc_body(part_ref, vn_ref, inv_ref, w_ref, b_ref, out_ref):
    seg_sum = jnp.sum(part_ref[...], axis=0)
    pool = seg_sum * inv_ref[...]
    x = vn_ref[...] + pool
    y = jnp.dot(x, w_ref[...], preferred_element_type=jnp.float32) + b_ref[...]
    out_ref[...] = vn_ref[...] + jnp.maximum(y, 0.0)


_fc = pl.pallas_call(
    _fc_body,
    out_shape=jax.ShapeDtypeStruct((B, D), jnp.float32),
)


def _broadcast_body(h_hbm, vn_hbm, bnd_hbm, out_hbm, bnd_v, buf, tab):
    wid = _worker_id()
    pltpu.sync_copy(bnd_hbm, bnd_v)
    pltpu.sync_copy(vn_hbm, tab)
    bnd = _load_bounds(bnd_v)

    def chunk_body(m, carry):
        t = wid + m * NW

        @pl.when(t < NCHUNK)
        def _():
            c0 = t * CH
            pltpu.sync_copy(h_hbm.at[pl.ds(c0, CH)], buf)
            for s in range(B):
                lo = jnp.maximum(bnd[s], c0)
                hi = jnp.minimum(bnd[s + 1], c0 + CH)

                @pl.when(lo < hi)
                def _():
                    t8 = tuple(
                        tab[s, pl.ds(NLANE * j, NLANE)] for j in range(NJ)
                    )

                    def row_body(i, c):
                        off = i - c0
                        for j in range(NJ):
                            sl = pl.ds(NLANE * j, NLANE)
                            buf[off, sl] = buf[off, sl] + t8[j]
                        return c

                    lax.fori_loop(lo, hi, row_body, 0)

            pltpu.sync_copy(buf, out_hbm.at[pl.ds(c0, CH)])

        return carry

    lax.fori_loop(0, CH_PER_W, chunk_body, 0)


_broadcast = pl.kernel(
    _broadcast_body,
    out_type=jax.ShapeDtypeStruct((N, D), jnp.float32),
    mesh=_mesh,
    scratch_types=[
        pltpu.VMEM((BND_PAD,), jnp.int32),
        pltpu.VMEM((CH, D), jnp.float32),
        pltpu.VMEM((B, D), jnp.float32),
    ],
)


@jax.jit
def kernel(h, vn_h, segment_ids, W, b):
    bnd = jnp.searchsorted(
        segment_ids, jnp.arange(B + 1, dtype=segment_ids.dtype)
    ).astype(jnp.int32)
    bnd_pad = jnp.zeros((BND_PAD,), jnp.int32).at[: B + 1].set(bnd)

    part = _seg_partial(h, bnd_pad)

    counts = jnp.maximum((bnd[1:] - bnd[:-1]).astype(jnp.float32), 1.0)
    inv = (1.0 / counts)[:, None]
    vn_h_new = _fc(part, vn_h, inv, W, b[None, :])

    h_new = _broadcast(h, vn_h_new, bnd_pad)
    return (vn_h_new, h_new)
